# Initial kernel scaffold; baseline (speedup 1.0000x reference)
import jax
import jax.numpy as jnp
from jax.experimental import pallas as pl


def kernel(word_ids, topic_ids, wd_src, wd_dst, ww_src, ww_dst, wt_src, wt_dst, td_src, td_dst, tt_src, tt_dst, y_data, word_embeds, topic_embeds, W_wt, b_wt, W_ww, b_ww, W_wd, b_wd, W_td, b_td, W_tt, b_tt, out_W, out_b):
    raise NotImplementedError("write your pallas kernel here")


# speedup vs baseline: 19.4482x; 19.4482x over previous
import jax
import jax.numpy as jnp
from jax import lax
from jax.experimental import pallas as pl
from jax.experimental.pallas import tpu as pltpu
from jax.experimental.pallas import tpu_sc as plsc

N_WORD = 40000
N_TOPIC = 800
N_DOC = 3200
B = 16
H = 128
VOCAB = 15000
NUM_TOPIC = 50

OW = 40960
OT = 1024
ND = 3328
E_WD, P_WD = 200000, 200704
E_TD, P_TD = 20000, 20480


def _rsqrt16(x):
    i = lax.bitcast_convert_type(x, jnp.int32)
    i = jnp.int32(0x5F3759DF) - lax.shift_right_arithmetic(i, 1)
    y = lax.bitcast_convert_type(i, jnp.float32)
    for _ in range(3):
        y = y * (1.5 - 0.5 * x * y * y)
    return y


def _k1_body(we_ref, te_ref, wwd_ref, wtd_ref, ow_ref, uws_ref, uts_ref):
    ow = ow_ref[...]
    vw = jnp.dot(wwd_ref[...], ow, preferred_element_type=jnp.float32)
    vt = jnp.dot(wtd_ref[...], ow, preferred_element_type=jnp.float32)
    uws_ref[...] = jnp.dot(we_ref[...], vw, preferred_element_type=jnp.float32)
    uts_ref[...] = jnp.dot(te_ref[...], vt, preferred_element_type=jnp.float32)


def _sc_body(wdS, wdD, tdS, tdD, widp, tidp, uws, uts, out,
             cnt_ow, cnt_iw, cnt_ot, cnt_it, m_wd, m_td, nv_w, nv_t,
             zbuf, ones, ones_t, ibuf, tbuf, ibuf2, tbuf2, vbuf, vbuf_t,
             gidx, gbuf, cbuf, nbuf, gidx_t, gbuf_t, cbuf_t, nbuf_t,
             mw, mt, ciw, cit, accbuf, sem):
    c = lax.axis_index("c")
    s = lax.axis_index("s")

    def _z16(i, _):
        zbuf[pl.ds(i * 16, 16)] = jnp.zeros((16,), jnp.float32)
        return 0
    lax.fori_loop(0, 160, _z16, 0)

    def _o16(i, _):
        ones[pl.ds(i * 16, 16)] = jnp.ones((16,), jnp.float32)
        return 0
    lax.fori_loop(0, 784, _o16, 0)

    def _ot16(i, _):
        ones_t[pl.ds(i * 16, 16)] = jnp.ones((16,), jnp.float32)
        return 0
    lax.fori_loop(0, 80, _ot16, 0)

    pltpu.sync_copy(zbuf, cnt_ow.at[pl.ds(s * 2560, 2560)])
    pltpu.sync_copy(zbuf.at[pl.ds(0, 208)], cnt_iw.at[pl.ds(s * 208, 208)])
    pltpu.sync_copy(zbuf.at[pl.ds(0, 64)], cnt_ot.at[pl.ds(s * 64, 64)])
    pltpu.sync_copy(zbuf.at[pl.ds(0, 208)], cnt_it.at[pl.ds(s * 208, 208)])
    pltpu.sync_copy(zbuf.at[pl.ds(0, 208)], m_wd.at[pl.ds(s * 208, 208)])
    pltpu.sync_copy(zbuf.at[pl.ds(0, 208)], m_td.at[pl.ds(s * 208, 208)])
    plsc.subcore_barrier()

    pltpu.sync_copy(wdS.at[pl.ds(s * 12544, 12544)], ibuf)
    pltpu.sync_copy(ones, cnt_ow.at[ibuf], add=True)
    pltpu.sync_copy(wdD.at[pl.ds(s * 12544, 12544)], ibuf)
    pltpu.sync_copy(ones, cnt_iw.at[ibuf], add=True)
    pltpu.sync_copy(tdS.at[pl.ds(s * 1280, 1280)], tbuf)
    pltpu.sync_copy(ones_t, cnt_ot.at[tbuf], add=True)
    pltpu.sync_copy(tdD.at[pl.ds(s * 1280, 1280)], tbuf)
    pltpu.sync_copy(ones_t, cnt_it.at[tbuf], add=True)
    plsc.subcore_barrier()

    pltpu.sync_copy(widp.at[pl.ds(s * 2560, 2560)], gidx)
    pltpu.async_copy(uws.at[gidx], gbuf, sem).wait()
    pltpu.sync_copy(cnt_ow.at[pl.ds(s * 2560, 2560)], cbuf)

    def _nv16(i, _):
        cnt = jnp.maximum(cbuf[pl.ds(i * 16, 16)], 1.0)
        nbuf[pl.ds(i * 16, 16)] = _rsqrt16(cnt) * gbuf[pl.ds(i * 16, 16)]
        return 0
    lax.fori_loop(0, 160, _nv16, 0)
    pltpu.sync_copy(nbuf, nv_w.at[pl.ds(s * 2560, 2560)])

    pltpu.sync_copy(tidp.at[pl.ds(s * 64, 64)], gidx_t)
    pltpu.async_copy(uts.at[gidx_t], gbuf_t, sem).wait()
    pltpu.sync_copy(cnt_ot.at[pl.ds(s * 64, 64)], cbuf_t)

    def _nvt16(i, _):
        cnt = jnp.maximum(cbuf_t[pl.ds(i * 16, 16)], 1.0)
        nbuf_t[pl.ds(i * 16, 16)] = _rsqrt16(cnt) * gbuf_t[pl.ds(i * 16, 16)]
        return 0
    lax.fori_loop(0, 4, _nvt16, 0)
    pltpu.sync_copy(nbuf_t, nv_t.at[pl.ds(s * 64, 64)])
    plsc.subcore_barrier()

    w = c * 16 + s
    pltpu.sync_copy(wdS.at[pl.ds(w * 6272, 6272)], ibuf2)
    pltpu.async_copy(nv_w.at[ibuf2], vbuf, sem).wait()
    pltpu.sync_copy(wdD.at[pl.ds(w * 6272, 6272)], ibuf2)
    pltpu.sync_copy(vbuf, m_wd.at[ibuf2], add=True)

    pltpu.sync_copy(tdS.at[pl.ds(w * 640, 640)], tbuf2)
    pltpu.async_copy(nv_t.at[tbuf2], vbuf_t, sem).wait()
    pltpu.sync_copy(tdD.at[pl.ds(w * 640, 640)], tbuf2)
    pltpu.sync_copy(vbuf_t, m_td.at[tbuf2], add=True)
    plsc.subcore_barrier()

    base = s * 200
    pltpu.sync_copy(m_wd.at[pl.ds(base, 208)], mw)
    pltpu.sync_copy(m_td.at[pl.ds(base, 208)], mt)
    pltpu.sync_copy(cnt_iw.at[pl.ds(base, 208)], ciw)
    pltpu.sync_copy(cnt_it.at[pl.ds(base, 208)], cit)
    lane = lax.iota(jnp.int32, 16)

    def _dot16(j, acc):
        cw = jnp.maximum(ciw[pl.ds(j * 16, 16)], 1.0)
        ct = jnp.maximum(cit[pl.ds(j * 16, 16)], 1.0)
        v = mw[pl.ds(j * 16, 16)] * _rsqrt16(cw) + mt[pl.ds(j * 16, 16)] * _rsqrt16(ct)
        return acc + jnp.where(j * 16 + lane < 200, v, 0.0)
    acc = lax.fori_loop(0, 13, _dot16, jnp.zeros((16,), jnp.float32))
    accbuf[...] = acc
    pltpu.sync_copy(accbuf, out.at[pl.ds(w * 16, 16)])


def _k3_body(p_ref, y_ref, bw_ref, bt_ref, ow_ref, ob_ref, loss_ref, pred_ref):
    dsum = jnp.sum(p_ref[...], axis=0, keepdims=True)
    bias = jnp.sum((bw_ref[...] + bt_ref[...]) * ow_ref[...].T) + ob_ref[0, 0]
    logits = dsum / 200.0 + bias
    y = y_ref[...]
    loss_ref[...] = jnp.mean(
        jnp.maximum(logits, 0.0) - logits * y
        + jnp.log(1.0 + jnp.exp(-jnp.abs(logits))), keepdims=True).reshape(1, 1)
    pred_ref[...] = 1.0 / (1.0 + jnp.exp(-logits))


def kernel(word_ids, topic_ids, wd_src, wd_dst, ww_src, ww_dst, wt_src, wt_dst,
           td_src, td_dst, tt_src, tt_dst, y_data, word_embeds, topic_embeds,
           W_wt, b_wt, W_ww, b_ww, W_wd, b_wd, W_td, b_td, W_tt, b_tt, out_W, out_b):
    f32, i32 = jnp.float32, jnp.int32

    uws, uts = pl.pallas_call(
        _k1_body,
        out_shape=[jax.ShapeDtypeStruct((VOCAB, 1), f32),
                   jax.ShapeDtypeStruct((NUM_TOPIC, 1), f32)],
    )(word_embeds, topic_embeds, W_wd, W_td, out_W)
    uws = uws.reshape(VOCAB)
    uts = uts.reshape(NUM_TOPIC)

    wdS = jnp.concatenate([wd_src, jnp.full((P_WD - E_WD,), OW - 1, i32)])
    wdD = jnp.concatenate([wd_dst, jnp.full((P_WD - E_WD,), ND - 1, i32)])
    tdS = jnp.concatenate([td_src, jnp.full((P_TD - E_TD,), OT - 1, i32)])
    tdD = jnp.concatenate([td_dst, jnp.full((P_TD - E_TD,), ND - 1, i32)])
    widp = jnp.concatenate([word_ids, jnp.zeros((OW - N_WORD,), i32)])
    tidp = jnp.concatenate([topic_ids, jnp.zeros((OT - N_TOPIC,), i32)])

    mesh = plsc.VectorSubcoreMesh(core_axis_name="c", subcore_axis_name="s")
    partial = pl.kernel(
        _sc_body,
        out_type=jax.ShapeDtypeStruct((512,), f32),
        mesh=mesh,
        scratch_types=[
            pltpu.VMEM_SHARED((OW,), f32),
            pltpu.VMEM_SHARED((ND,), f32),
            pltpu.VMEM_SHARED((OT,), f32),
            pltpu.VMEM_SHARED((ND,), f32),
            pltpu.VMEM_SHARED((ND,), f32),
            pltpu.VMEM_SHARED((ND,), f32),
            pltpu.VMEM_SHARED((OW,), f32),
            pltpu.VMEM_SHARED((OT,), f32),
            pltpu.VMEM((2560,), f32),
            pltpu.VMEM((12544,), f32),
            pltpu.VMEM((1280,), f32),
            pltpu.VMEM((12544,), i32),
            pltpu.VMEM((1280,), i32),
            pltpu.VMEM((6272,), i32),
            pltpu.VMEM((640,), i32),
            pltpu.VMEM((6272,), f32),
            pltpu.VMEM((640,), f32),
            pltpu.VMEM((2560,), i32),
            pltpu.VMEM((2560,), f32),
            pltpu.VMEM((2560,), f32),
            pltpu.VMEM((2560,), f32),
            pltpu.VMEM((64,), i32),
            pltpu.VMEM((64,), f32),
            pltpu.VMEM((64,), f32),
            pltpu.VMEM((64,), f32),
            pltpu.VMEM((208,), f32),
            pltpu.VMEM((208,), f32),
            pltpu.VMEM((208,), f32),
            pltpu.VMEM((208,), f32),
            pltpu.VMEM((16,), f32),
            pltpu.SemaphoreType.DMA,
        ],
    )(wdS, wdD, tdS, tdD, widp, tidp, uws, uts)

    loss, pred = pl.pallas_call(
        _k3_body,
        out_shape=[jax.ShapeDtypeStruct((1, 1), f32),
                   jax.ShapeDtypeStruct((1, B), f32)],
    )(partial.reshape(32, 16), y_data.reshape(1, B), b_wd.reshape(1, H),
      b_td.reshape(1, H), out_W, out_b.reshape(1, 1))

    return loss.reshape(()), pred.reshape(B, 1)
